# Initial kernel scaffold; baseline (speedup 1.0000x reference)
import jax
import jax.numpy as jnp
from jax import lax
from jax.experimental import pallas as pl
from jax.experimental.pallas import tpu as pltpu
from jax.experimental.pallas import tpu_sc as plsc

N_NODES = 10000
N_EDGES = 320000
D = 128

NC = 2
NS = 16
CHUNK = 80
EDGES_PER_TILE = N_EDGES // (NC * NS)
N_CHUNKS = EDGES_PER_TILE // CHUNK
ROWS_PER_TILE = N_NODES // NS
STAGE_ROWS = 125


def _seg_sum_kernel(temp_hbm, src_hbm, dst_hbm, zeros_hbm, out_hbm,
                    src_v, dst_v, rows_v, stage_v, table_sh, sem):
    c = lax.axis_index("c")
    s = lax.axis_index("s")

    pltpu.sync_copy(zeros_hbm, stage_v)
    for k in range(ROWS_PER_TILE // STAGE_ROWS):
        pltpu.sync_copy(stage_v,
                        table_sh.at[pl.ds(s * ROWS_PER_TILE + k * STAGE_ROWS,
                                          STAGE_ROWS)])
    plsc.subcore_barrier()

    edge_base = (c * NS + s) * EDGES_PER_TILE

    def body(i, carry):
        base = edge_base + i * CHUNK
        pltpu.sync_copy(src_hbm.at[pl.ds(base, CHUNK)], src_v)
        pltpu.sync_copy(dst_hbm.at[pl.ds(base, CHUNK)], dst_v)
        pltpu.async_copy(temp_hbm.at[src_v], rows_v, sem).wait()
        pltpu.sync_copy(rows_v, table_sh.at[dst_v], add=True)
        return carry

    lax.fori_loop(0, N_CHUNKS, body, 0)
    plsc.subcore_barrier()

    for k in range(ROWS_PER_TILE // STAGE_ROWS):
        r0 = s * ROWS_PER_TILE + k * STAGE_ROWS
        pltpu.sync_copy(table_sh.at[pl.ds(r0, STAGE_ROWS)], stage_v)
        pltpu.sync_copy(stage_v, out_hbm.at[c, pl.ds(r0, STAGE_ROWS)])


def _seg_sum(temp, src, dst, zeros_stage):
    mesh = plsc.VectorSubcoreMesh(core_axis_name="c", subcore_axis_name="s",
                                  num_cores=NC, num_subcores=NS)
    kern = pl.kernel(
        _seg_sum_kernel,
        out_type=jax.ShapeDtypeStruct((NC, N_NODES, D), jnp.float32),
        mesh=mesh,
        scratch_types=[
            pltpu.VMEM((CHUNK,), jnp.int32),
            pltpu.VMEM((CHUNK,), jnp.int32),
            pltpu.VMEM((CHUNK, D), jnp.float32),
            pltpu.VMEM((STAGE_ROWS, D), jnp.float32),
            pltpu.VMEM_SHARED((N_NODES, D), jnp.float32),
            pltpu.SemaphoreType.DMA,
        ],
    )
    return kern(temp, src, dst, zeros_stage)


ROW_BLK = 1000


def _lin_kernel(t_ref, a0_ref, a1_ref, w_ref, b_ref, sc_ref, o_ref):
    x = sc_ref[...] * t_ref[...] + a0_ref[0] + a1_ref[0]
    y = jnp.dot(x, w_ref[...], preferred_element_type=jnp.float32)
    o_ref[...] = jnp.maximum(y + b_ref[...], 0.0)


def _lin_layer(temp, agg2, w, b_row, scale_row):
    grid = (N_NODES // ROW_BLK,)
    return pl.pallas_call(
        _lin_kernel,
        grid=grid,
        in_specs=[
            pl.BlockSpec((ROW_BLK, D), lambda i: (i, 0)),
            pl.BlockSpec((1, ROW_BLK, D), lambda i: (0, i, 0)),
            pl.BlockSpec((1, ROW_BLK, D), lambda i: (1, i, 0)),
            pl.BlockSpec((D, D), lambda i: (0, 0)),
            pl.BlockSpec((1, D), lambda i: (0, 0)),
            pl.BlockSpec((1, D), lambda i: (0, 0)),
        ],
        out_specs=pl.BlockSpec((ROW_BLK, D), lambda i: (i, 0)),
        out_shape=jax.ShapeDtypeStruct((N_NODES, D), jnp.float32),
    )(temp, agg2, agg2, w, b_row, scale_row)


def _lin2_kernel(t_ref, a0_ref, a1_ref, w_ref, b_ref, sc_ref, o_ref, acc_ref):
    i = pl.program_id(0)

    @pl.when(i == 0)
    def _():
        acc_ref[...] = jnp.zeros_like(acc_ref)

    x = sc_ref[...] * t_ref[...] + a0_ref[0] + a1_ref[0]
    y = jnp.dot(x, w_ref[...], preferred_element_type=jnp.float32)
    y = jnp.maximum(y + b_ref[...], 0.0)
    acc_ref[...] += jnp.sum(y, axis=0, keepdims=True)

    @pl.when(i == pl.num_programs(0) - 1)
    def _():
        o_ref[...] = jnp.maximum(acc_ref[...] * (1.0 / N_NODES), 0.0)


def _lin_layer2_readout(temp, agg2, w, b_row, scale_row):
    grid = (N_NODES // ROW_BLK,)
    return pl.pallas_call(
        _lin2_kernel,
        grid=grid,
        in_specs=[
            pl.BlockSpec((ROW_BLK, D), lambda i: (i, 0)),
            pl.BlockSpec((1, ROW_BLK, D), lambda i: (0, i, 0)),
            pl.BlockSpec((1, ROW_BLK, D), lambda i: (1, i, 0)),
            pl.BlockSpec((D, D), lambda i: (0, 0)),
            pl.BlockSpec((1, D), lambda i: (0, 0)),
            pl.BlockSpec((1, D), lambda i: (0, 0)),
        ],
        out_specs=pl.BlockSpec((1, D), lambda i: (0, 0)),
        out_shape=jax.ShapeDtypeStruct((1, D), jnp.float32),
        scratch_shapes=[pltpu.VMEM((1, D), jnp.float32)],
    )(temp, agg2, agg2, w, b_row, scale_row)


def kernel(X, h, epsilon, edge_index, W1, b1, W2, b2, eps0, eps1):
    temp = jnp.concatenate([X, epsilon, h], axis=1)
    src = edge_index[0]
    dst = edge_index[1]
    zeros_stage = jnp.zeros((STAGE_ROWS, D), jnp.float32)
    b1r = b1.reshape(1, D)
    b2r = b2.reshape(1, D)
    sc0 = jnp.full((1, D), 1.0, jnp.float32) + eps0
    sc1 = jnp.full((1, D), 1.0, jnp.float32) + eps1

    agg2 = _seg_sum(temp, src, dst, zeros_stage)
    y1 = _lin_layer(temp, agg2, W1, b1r, sc0)
    agg2 = _seg_sum(y1, src, dst, zeros_stage)
    out = _lin_layer2_readout(y1, agg2, W2, b2r, sc1)
    return (out.reshape(D), epsilon)


# speedup vs baseline: 5.0106x; 5.0106x over previous
import jax
import jax.numpy as jnp
from jax import lax
from jax.experimental import pallas as pl
from jax.experimental.pallas import tpu as pltpu
from jax.experimental.pallas import tpu_sc as plsc

N_NODES = 10000
N_EDGES = 320000
D = 128

NC = 2
NS = 16
CHUNK = 80
EDGES_PER_TILE = N_EDGES // (NC * NS)
N_CHUNKS = EDGES_PER_TILE // CHUNK
TABLE_ROWS = 10240
ROWS_PER_TILE = TABLE_ROWS // NS
STAGE_ROWS = 128


def _seg_sum_kernel(temp_hbm, src_hbm, dst_hbm, zeros_hbm, out_hbm,
                    src_v, dst_v, rows_v, stage_v, table_sh, sem):
    c = lax.axis_index("c")
    s = lax.axis_index("s")

    pltpu.sync_copy(zeros_hbm, stage_v)
    for k in range(ROWS_PER_TILE // STAGE_ROWS):
        pltpu.sync_copy(stage_v,
                        table_sh.at[pl.ds(s * ROWS_PER_TILE + k * STAGE_ROWS,
                                          STAGE_ROWS)])
    plsc.subcore_barrier()

    edge_base = (c * NS + s) * EDGES_PER_TILE

    def body(i, carry):
        base = edge_base + i * CHUNK
        pltpu.sync_copy(src_hbm.at[pl.ds(base, CHUNK)], src_v)
        pltpu.sync_copy(dst_hbm.at[pl.ds(base, CHUNK)], dst_v)
        pltpu.async_copy(temp_hbm.at[src_v], rows_v, sem).wait()
        pltpu.sync_copy(rows_v, table_sh.at[dst_v], add=True)
        return carry

    lax.fori_loop(0, N_CHUNKS, body, 0)
    plsc.subcore_barrier()

    for k in range(ROWS_PER_TILE // STAGE_ROWS):
        r0 = s * ROWS_PER_TILE + k * STAGE_ROWS
        pltpu.sync_copy(table_sh.at[pl.ds(r0, STAGE_ROWS)], stage_v)
        pltpu.sync_copy(stage_v, out_hbm.at[c, pl.ds(r0, STAGE_ROWS)])


def _seg_sum(temp, src, dst, zeros_stage):
    mesh = plsc.VectorSubcoreMesh(core_axis_name="c", subcore_axis_name="s",
                                  num_cores=NC, num_subcores=NS)
    kern = pl.kernel(
        _seg_sum_kernel,
        out_type=jax.ShapeDtypeStruct((NC, TABLE_ROWS, D), jnp.float32),
        mesh=mesh,
        scratch_types=[
            pltpu.VMEM((CHUNK,), jnp.int32),
            pltpu.VMEM((CHUNK,), jnp.int32),
            pltpu.VMEM((CHUNK, D), jnp.float32),
            pltpu.VMEM((STAGE_ROWS, D), jnp.float32),
            pltpu.VMEM_SHARED((TABLE_ROWS, D), jnp.float32),
            pltpu.SemaphoreType.DMA,
        ],
    )
    return kern(temp, src, dst, zeros_stage)


ROW_BLK = 1000


def _lin_kernel(t_ref, a0_ref, a1_ref, w_ref, b_ref, sc_ref, o_ref):
    x = sc_ref[...] * t_ref[...] + a0_ref[0] + a1_ref[0]
    y = jnp.dot(x, w_ref[...], preferred_element_type=jnp.float32)
    o_ref[...] = jnp.maximum(y + b_ref[...], 0.0)


def _lin_layer(temp, agg2, w, b_row, scale_row):
    grid = (N_NODES // ROW_BLK,)
    return pl.pallas_call(
        _lin_kernel,
        grid=grid,
        in_specs=[
            pl.BlockSpec((ROW_BLK, D), lambda i: (i, 0)),
            pl.BlockSpec((1, ROW_BLK, D), lambda i: (0, i, 0)),
            pl.BlockSpec((1, ROW_BLK, D), lambda i: (1, i, 0)),
            pl.BlockSpec((D, D), lambda i: (0, 0)),
            pl.BlockSpec((1, D), lambda i: (0, 0)),
            pl.BlockSpec((1, D), lambda i: (0, 0)),
        ],
        out_specs=pl.BlockSpec((ROW_BLK, D), lambda i: (i, 0)),
        out_shape=jax.ShapeDtypeStruct((N_NODES, D), jnp.float32),
    )(temp, agg2, agg2, w, b_row, scale_row)


def _lin2_kernel(t_ref, a0_ref, a1_ref, w_ref, b_ref, sc_ref, o_ref, acc_ref):
    i = pl.program_id(0)

    @pl.when(i == 0)
    def _():
        acc_ref[...] = jnp.zeros_like(acc_ref)

    x = sc_ref[...] * t_ref[...] + a0_ref[0] + a1_ref[0]
    y = jnp.dot(x, w_ref[...], preferred_element_type=jnp.float32)
    y = jnp.maximum(y + b_ref[...], 0.0)
    acc_ref[...] += jnp.sum(y, axis=0, keepdims=True)

    @pl.when(i == pl.num_programs(0) - 1)
    def _():
        o_ref[...] = jnp.maximum(acc_ref[...] * (1.0 / N_NODES), 0.0)


def _lin_layer2_readout(temp, agg2, w, b_row, scale_row):
    grid = (N_NODES // ROW_BLK,)
    return pl.pallas_call(
        _lin2_kernel,
        grid=grid,
        in_specs=[
            pl.BlockSpec((ROW_BLK, D), lambda i: (i, 0)),
            pl.BlockSpec((1, ROW_BLK, D), lambda i: (0, i, 0)),
            pl.BlockSpec((1, ROW_BLK, D), lambda i: (1, i, 0)),
            pl.BlockSpec((D, D), lambda i: (0, 0)),
            pl.BlockSpec((1, D), lambda i: (0, 0)),
            pl.BlockSpec((1, D), lambda i: (0, 0)),
        ],
        out_specs=pl.BlockSpec((1, D), lambda i: (0, 0)),
        out_shape=jax.ShapeDtypeStruct((1, D), jnp.float32),
        scratch_shapes=[pltpu.VMEM((1, D), jnp.float32)],
    )(temp, agg2, agg2, w, b_row, scale_row)


def kernel(X, h, epsilon, edge_index, W1, b1, W2, b2, eps0, eps1):
    temp = jnp.concatenate([X, epsilon, h], axis=1)
    src = edge_index[0]
    dst = edge_index[1]
    zeros_stage = jnp.zeros((STAGE_ROWS, D), jnp.float32)
    b1r = b1.reshape(1, D)
    b2r = b2.reshape(1, D)
    sc0 = jnp.full((1, D), 1.0, jnp.float32) + eps0
    sc1 = jnp.full((1, D), 1.0, jnp.float32) + eps1

    agg2 = _seg_sum(temp, src, dst, zeros_stage)
    y1 = _lin_layer(temp, agg2, W1, b1r, sc0)
    agg2 = _seg_sum(y1, src, dst, zeros_stage)
    out = _lin_layer2_readout(y1, agg2, W2, b2r, sc1)
    return (out.reshape(D), epsilon)


# speedup vs baseline: 10.9926x; 2.1939x over previous
import jax
import jax.numpy as jnp
from jax import lax
from jax.experimental import pallas as pl
from jax.experimental.pallas import tpu as pltpu
from jax.experimental.pallas import tpu_sc as plsc

N_NODES = 10000
N_EDGES = 320000
D = 128

NC = 2
NS = 16
CHUNK = 80
EDGES_PER_TILE = N_EDGES // (NC * NS)
N_CHUNKS = EDGES_PER_TILE // CHUNK
TABLE_ROWS = 10240
ROWS_PER_TILE = TABLE_ROWS // NS


def _seg_sum_kernel(temp_hbm, src_hbm, dst_hbm, zeros_hbm, out_hbm,
                    src_slab, dstb0, dstb1, rows0, rows1, table_sh,
                    sem0, sem1):
    c = lax.axis_index("c")
    s = lax.axis_index("s")
    wid = c * NS + s

    pltpu.sync_copy(src_hbm.at[wid], src_slab)

    pltpu.sync_copy(zeros_hbm, rows0)
    for k in range(ROWS_PER_TILE // CHUNK):
        pltpu.sync_copy(rows0,
                        table_sh.at[pl.ds(s * ROWS_PER_TILE + k * CHUNK,
                                          CHUNK)])
    plsc.subcore_barrier()

    rows = (rows0, rows1)
    dstb = (dstb0, dstb1)
    sems = (sem0, sem1)

    for b in range(2):
        pltpu.async_copy(dst_hbm.at[wid, b], dstb[b], sems[b])
        pltpu.async_copy(temp_hbm.at[src_slab.at[b]], rows[b], sems[b])

    def wait_chunk(b):
        pltpu.make_async_copy(dst_hbm.at[wid, 0], dstb[b], sems[b]).wait()
        pltpu.make_async_copy(temp_hbm.at[pl.ds(0, CHUNK)],
                              rows[b], sems[b]).wait()

    def pair(k, carry):
        for b in range(2):
            i = 2 * k + b
            wait_chunk(b)
            pltpu.sync_copy(rows[b], table_sh.at[dstb[b]], add=True)

            @pl.when(i + 2 < N_CHUNKS)
            def _():
                pltpu.async_copy(dst_hbm.at[wid, i + 2], dstb[b], sems[b])
                pltpu.async_copy(temp_hbm.at[src_slab.at[i + 2]],
                                 rows[b], sems[b])
        return carry

    lax.fori_loop(0, N_CHUNKS // 2, pair, 0)
    wait_chunk(0)
    pltpu.sync_copy(rows0, table_sh.at[dstb0], add=True)

    plsc.subcore_barrier()

    for k in range(ROWS_PER_TILE // CHUNK):
        r0 = s * ROWS_PER_TILE + k * CHUNK
        pltpu.sync_copy(table_sh.at[pl.ds(r0, CHUNK)], rows0)
        pltpu.sync_copy(rows0, out_hbm.at[c, pl.ds(r0, CHUNK)])


def _seg_sum(temp, src3, dst3, zeros_stage):
    mesh = plsc.VectorSubcoreMesh(core_axis_name="c", subcore_axis_name="s",
                                  num_cores=NC, num_subcores=NS)
    kern = pl.kernel(
        _seg_sum_kernel,
        out_type=jax.ShapeDtypeStruct((NC, TABLE_ROWS, D), jnp.float32),
        mesh=mesh,
        scratch_types=[
            pltpu.VMEM((N_CHUNKS, CHUNK), jnp.int32),
            pltpu.VMEM((CHUNK,), jnp.int32),
            pltpu.VMEM((CHUNK,), jnp.int32),
            pltpu.VMEM((CHUNK, D), jnp.float32),
            pltpu.VMEM((CHUNK, D), jnp.float32),
            pltpu.VMEM_SHARED((TABLE_ROWS, D), jnp.float32),
            pltpu.SemaphoreType.DMA,
            pltpu.SemaphoreType.DMA,
        ],
    )
    return kern(temp, src3, dst3, zeros_stage)


ROW_BLK = 1000


def _lin_kernel(t_ref, a0_ref, a1_ref, w_ref, b_ref, sc_ref, o_ref):
    x = sc_ref[...] * t_ref[...] + a0_ref[0] + a1_ref[0]
    y = jnp.dot(x, w_ref[...], preferred_element_type=jnp.float32)
    o_ref[...] = jnp.maximum(y + b_ref[...], 0.0)


def _lin_layer(temp, agg2, w, b_row, scale_row):
    grid = (N_NODES // ROW_BLK,)
    return pl.pallas_call(
        _lin_kernel,
        grid=grid,
        in_specs=[
            pl.BlockSpec((ROW_BLK, D), lambda i: (i, 0)),
            pl.BlockSpec((1, ROW_BLK, D), lambda i: (0, i, 0)),
            pl.BlockSpec((1, ROW_BLK, D), lambda i: (1, i, 0)),
            pl.BlockSpec((D, D), lambda i: (0, 0)),
            pl.BlockSpec((1, D), lambda i: (0, 0)),
            pl.BlockSpec((1, D), lambda i: (0, 0)),
        ],
        out_specs=pl.BlockSpec((ROW_BLK, D), lambda i: (i, 0)),
        out_shape=jax.ShapeDtypeStruct((N_NODES, D), jnp.float32),
    )(temp, agg2, agg2, w, b_row, scale_row)


def _lin2_kernel(t_ref, a0_ref, a1_ref, w_ref, b_ref, sc_ref, o_ref, acc_ref):
    i = pl.program_id(0)

    @pl.when(i == 0)
    def _():
        acc_ref[...] = jnp.zeros_like(acc_ref)

    x = sc_ref[...] * t_ref[...] + a0_ref[0] + a1_ref[0]
    y = jnp.dot(x, w_ref[...], preferred_element_type=jnp.float32)
    y = jnp.maximum(y + b_ref[...], 0.0)
    acc_ref[...] += jnp.sum(y, axis=0, keepdims=True)

    @pl.when(i == pl.num_programs(0) - 1)
    def _():
        o_ref[...] = jnp.maximum(acc_ref[...] * (1.0 / N_NODES), 0.0)


def _lin_layer2_readout(temp, agg2, w, b_row, scale_row):
    grid = (N_NODES // ROW_BLK,)
    return pl.pallas_call(
        _lin2_kernel,
        grid=grid,
        in_specs=[
            pl.BlockSpec((ROW_BLK, D), lambda i: (i, 0)),
            pl.BlockSpec((1, ROW_BLK, D), lambda i: (0, i, 0)),
            pl.BlockSpec((1, ROW_BLK, D), lambda i: (1, i, 0)),
            pl.BlockSpec((D, D), lambda i: (0, 0)),
            pl.BlockSpec((1, D), lambda i: (0, 0)),
            pl.BlockSpec((1, D), lambda i: (0, 0)),
        ],
        out_specs=pl.BlockSpec((1, D), lambda i: (0, 0)),
        out_shape=jax.ShapeDtypeStruct((1, D), jnp.float32),
        scratch_shapes=[pltpu.VMEM((1, D), jnp.float32)],
    )(temp, agg2, agg2, w, b_row, scale_row)


def kernel(X, h, epsilon, edge_index, W1, b1, W2, b2, eps0, eps1):
    temp = jnp.concatenate([X, epsilon, h], axis=1)
    src = edge_index[0].reshape(NC * NS, N_CHUNKS, CHUNK)
    dst = edge_index[1].reshape(NC * NS, N_CHUNKS, CHUNK)
    zeros_stage = jnp.zeros((CHUNK, D), jnp.float32)
    b1r = b1.reshape(1, D)
    b2r = b2.reshape(1, D)
    sc0 = jnp.full((1, D), 1.0, jnp.float32) + eps0
    sc1 = jnp.full((1, D), 1.0, jnp.float32) + eps1

    agg2 = _seg_sum(temp, src, dst, zeros_stage)
    y1 = _lin_layer(temp, agg2, W1, b1r, sc0)
    agg2 = _seg_sum(y1, src, dst, zeros_stage)
    out = _lin_layer2_readout(y1, agg2, W2, b2r, sc1)
    return (out.reshape(D), epsilon)


# speedup vs baseline: 12.4719x; 1.1346x over previous
import jax
import jax.numpy as jnp
from jax import lax
from jax.experimental import pallas as pl
from jax.experimental.pallas import tpu as pltpu
from jax.experimental.pallas import tpu_sc as plsc

N_NODES = 10000
N_EDGES = 320000
D = 128

NC = 2
NS = 16
CHUNK = 80
EDGES_PER_TILE = N_EDGES // (NC * NS)
N_CHUNKS = EDGES_PER_TILE // CHUNK
TABLE_ROWS = 10240
ROWS_PER_TILE = TABLE_ROWS // NS
NB = 3


def _seg_sum_kernel(temp_hbm, src_hbm, dst_hbm, zeros_hbm, out_hbm,
                    rows0, rows1, rows2, sb0, sb1, sb2, db0, db1, db2,
                    table_sh, g0, g1, g2, s0, s1, s2, x0, x1, x2):
    c = lax.axis_index("c")
    s = lax.axis_index("s")
    wid = c * NS + s

    rows = (rows0, rows1, rows2)
    sidx = (sb0, sb1, sb2)
    dstb = (db0, db1, db2)
    gsem = (g0, g1, g2)
    ssem = (s0, s1, s2)
    xsem = (x0, x1, x2)

    pltpu.sync_copy(zeros_hbm, rows0)
    for k in range(ROWS_PER_TILE // CHUNK):
        pltpu.sync_copy(rows0,
                        table_sh.at[pl.ds(s * ROWS_PER_TILE + k * CHUNK,
                                          CHUNK)])
    plsc.subcore_barrier()

    def load_src(i, b):
        pltpu.async_copy(src_hbm.at[wid, i], sidx[b], xsem[b])

    def wait_src(b):
        pltpu.make_async_copy(src_hbm.at[wid, 0], sidx[b], xsem[b]).wait()

    def load_dst(i, b):
        pltpu.async_copy(dst_hbm.at[wid, i], dstb[b], gsem[b])

    def issue_gather(b):
        pltpu.async_copy(temp_hbm.at[sidx[b]], rows[b], gsem[b])

    def wait_gather(b):
        pltpu.make_async_copy(dst_hbm.at[wid, 0], dstb[b], gsem[b]).wait()
        pltpu.make_async_copy(temp_hbm.at[pl.ds(0, CHUNK)],
                              rows[b], gsem[b]).wait()

    def start_scatter(b):
        pltpu.async_copy(rows[b], table_sh.at[dstb[b]], ssem[b], add=True)

    def wait_scatter(b):
        pltpu.make_async_copy(rows[b], table_sh.at[dstb[b]], ssem[b]).wait()

    for b in range(NB):
        load_src(b, b)
    for b in range(2):
        load_dst(b, b)
        wait_src(b)
        issue_gather(b)

    def step(i, b, first, last):
        wait_gather(b)
        start_scatter(b)
        if not first:
            wait_scatter((b + 2) % NB)
        if last:
            return
        bn = (b + 2) % NB

        def load_src_ahead():
            load_src(i + NB, b)

        def issue_ahead():
            load_dst(i + 2, bn)
            wait_src(bn)
            issue_gather(bn)

        if isinstance(i, int):
            if i + NB < N_CHUNKS:
                load_src_ahead()
            if i + 2 < N_CHUNKS:
                issue_ahead()
        else:
            pl.when(i + NB < N_CHUNKS)(load_src_ahead)
            pl.when(i + 2 < N_CHUNKS)(issue_ahead)

    step(0, 0, True, False)

    def triple(k, carry):
        i = 3 * k
        step(i + 1, 1, False, False)
        step(i + 2, 2, False, False)
        step(i + 3, 0, False, False)
        return carry

    lax.fori_loop(0, (N_CHUNKS - 2) // 3, triple, 0)
    step(N_CHUNKS - 1, (N_CHUNKS - 1) % NB, False, True)
    wait_scatter((N_CHUNKS - 1) % NB)

    plsc.subcore_barrier()

    for k in range(ROWS_PER_TILE // CHUNK):
        r0 = s * ROWS_PER_TILE + k * CHUNK
        pltpu.sync_copy(table_sh.at[pl.ds(r0, CHUNK)], rows0)
        pltpu.sync_copy(rows0, out_hbm.at[c, pl.ds(r0, CHUNK)])


def _seg_sum(temp, src3, dst3, zeros_stage):
    mesh = plsc.VectorSubcoreMesh(core_axis_name="c", subcore_axis_name="s",
                                  num_cores=NC, num_subcores=NS)
    kern = pl.kernel(
        _seg_sum_kernel,
        out_type=jax.ShapeDtypeStruct((NC, TABLE_ROWS, D), jnp.float32),
        mesh=mesh,
        scratch_types=(
            [pltpu.VMEM((CHUNK, D), jnp.float32) for _ in range(NB)]
            + [pltpu.VMEM((CHUNK,), jnp.int32) for _ in range(2 * NB)]
            + [pltpu.VMEM_SHARED((TABLE_ROWS, D), jnp.float32)]
            + [pltpu.SemaphoreType.DMA for _ in range(3 * NB)]
        ),
    )
    return kern(temp, src3, dst3, zeros_stage)


ROW_BLK = 1000


def _lin_body(t_ref, a0_ref, a1_ref, w_ref, b_ref, sc_ref):
    x = sc_ref[...] * t_ref[...] + a0_ref[0] + a1_ref[0]
    y = jnp.dot(x, w_ref[...], preferred_element_type=jnp.float32)
    return jnp.maximum(y + b_ref[...], 0.0)


def _lin_kernel(t_ref, a0_ref, a1_ref, w_ref, b_ref, sc_ref, o_ref):
    o_ref[...] = _lin_body(t_ref, a0_ref, a1_ref, w_ref, b_ref, sc_ref)


_LIN_IN_SPECS = [
    pl.BlockSpec((ROW_BLK, D), lambda i: (i, 0)),
    pl.BlockSpec((1, ROW_BLK, D), lambda i: (0, i, 0)),
    pl.BlockSpec((1, ROW_BLK, D), lambda i: (1, i, 0)),
    pl.BlockSpec((D, D), lambda i: (0, 0)),
    pl.BlockSpec((1, D), lambda i: (0, 0)),
    pl.BlockSpec((1, D), lambda i: (0, 0)),
]


def _lin_layer(temp, agg2, w, b_row, scale_row):
    return pl.pallas_call(
        _lin_kernel,
        grid=(N_NODES // ROW_BLK,),
        in_specs=_LIN_IN_SPECS,
        out_specs=pl.BlockSpec((ROW_BLK, D), lambda i: (i, 0)),
        out_shape=jax.ShapeDtypeStruct((N_NODES, D), jnp.float32),
    )(temp, agg2, agg2, w, b_row, scale_row)


def _lin2_kernel(t_ref, a0_ref, a1_ref, w_ref, b_ref, sc_ref, o_ref, acc_ref):
    i = pl.program_id(0)

    @pl.when(i == 0)
    def _():
        acc_ref[...] = jnp.zeros_like(acc_ref)

    y = _lin_body(t_ref, a0_ref, a1_ref, w_ref, b_ref, sc_ref)
    acc_ref[...] += jnp.sum(y, axis=0, keepdims=True)

    @pl.when(i == pl.num_programs(0) - 1)
    def _():
        o_ref[...] = jnp.maximum(acc_ref[...] * (1.0 / N_NODES), 0.0)


def _lin_layer2_readout(temp, agg2, w, b_row, scale_row):
    return pl.pallas_call(
        _lin2_kernel,
        grid=(N_NODES // ROW_BLK,),
        in_specs=_LIN_IN_SPECS,
        out_specs=pl.BlockSpec((1, D), lambda i: (0, 0)),
        out_shape=jax.ShapeDtypeStruct((1, D), jnp.float32),
        scratch_shapes=[pltpu.VMEM((1, D), jnp.float32)],
    )(temp, agg2, agg2, w, b_row, scale_row)


def kernel(X, h, epsilon, edge_index, W1, b1, W2, b2, eps0, eps1):
    temp = jnp.concatenate([X, epsilon, h], axis=1)
    src = edge_index[0].reshape(NC * NS, N_CHUNKS, CHUNK)
    dst = edge_index[1].reshape(NC * NS, N_CHUNKS, CHUNK)
    zeros_stage = jnp.zeros((CHUNK, D), jnp.float32)
    b1r = b1.reshape(1, D)
    b2r = b2.reshape(1, D)
    sc0 = jnp.full((1, D), 1.0, jnp.float32) + eps0
    sc1 = jnp.full((1, D), 1.0, jnp.float32) + eps1

    agg2 = _seg_sum(temp, src, dst, zeros_stage)
    y1 = _lin_layer(temp, agg2, W1, b1r, sc0)
    agg2 = _seg_sum(y1, src, dst, zeros_stage)
    out = _lin_layer2_readout(y1, agg2, W2, b2r, sc1)
    return (out.reshape(D), epsilon)
